# baseline (speedup 1.0000x reference)
import functools

import jax
import jax.numpy as jnp
from jax import lax
from jax.experimental import pallas as pl
from jax.experimental.pallas import tpu as pltpu
from jax.experimental.pallas import tpu_sc as plsc

N = 10000
E = 320000
D = 128
NLAB = 10

NC = 2
NS = 16
NW = NC * NS
EPW = E // NW
EPAD = 10112
CH = 64
NFULL = EPW // CH
NTAIL = EPW - NFULL * CH
SRCR = EPAD // 128
RPT = 624
TAIL = N - NS * RPT

BR = 1000
G = N // BR

_MESH = plsc.VectorSubcoreMesh(core_axis_name="c", subcore_axis_name="s")


def _init_acc(zeros_hbm, acc, s):
    pltpu.sync_copy(zeros_hbm.at[pl.ds(0, RPT)], acc.at[pl.ds(s * RPT, RPT)])

    @pl.when(s == NS - 1)
    def _():
        pltpu.sync_copy(zeros_hbm.at[pl.ds(0, TAIL)],
                        acc.at[pl.ds(NS * RPT, TAIL)])


def _dump_acc(acc, out_hbm, c, s):
    pltpu.sync_copy(acc.at[pl.ds(s * RPT, RPT)],
                    out_hbm.at[c, pl.ds(s * RPT, RPT)])

    @pl.when(s == NS - 1)
    def _():
        pltpu.sync_copy(acc.at[pl.ds(NS * RPT, TAIL)],
                        out_hbm.at[c, pl.ds(NS * RPT, TAIL)])


def _didx(dst_v, g):
    return dst_v.at[g // 2, pl.ds(CH * (g % 2), CH)]


@functools.partial(
    pl.kernel,
    out_type=jax.ShapeDtypeStruct((NC, N, D), jnp.float32),
    mesh=_MESH,
    scratch_types=[
        pltpu.VMEM((SRCR, 128), jnp.int32),
        pltpu.VMEM((8, 16), jnp.int32),
        pltpu.VMEM((CH, D), jnp.float32),
        pltpu.VMEM_SHARED((N, D), jnp.float32),
        pltpu.SemaphoreType.DMA,
        pltpu.SemaphoreType.DMA,
        pltpu.SemaphoreType.DMA,
        pltpu.SemaphoreType.DMA,
    ],
)
def _sc_degree(dst_hbm, dstt_hbm, ones_hbm, zeros_hbm, out_hbm,
               dst_v, dstt_v, ones_v, acc, s0, s1, s2, s3):
    c = lax.axis_index("c")
    s = lax.axis_index("s")
    wid = s * NC + c
    _init_acc(zeros_hbm, acc, s)
    pltpu.sync_copy(dst_hbm.at[wid], dst_v)
    pltpu.sync_copy(dstt_hbm.at[wid], dstt_v)
    pltpu.sync_copy(ones_hbm, ones_v)
    plsc.subcore_barrier()

    sems = (s0, s1, s2, s3)

    def body(j, carry):
        for b in range(4):
            pltpu.async_copy(ones_v, acc.at[_didx(dst_v, 4 * j + b)],
                             sems[b], add=True)
        for b in range(4):
            pltpu.make_async_copy(ones_v, acc.at[_didx(dst_v, 4 * j + b)],
                                  sems[b]).wait()
        return carry

    lax.fori_loop(0, NFULL // 4, body, 0)
    pltpu.sync_copy(ones_v.at[pl.ds(0, NTAIL)], acc.at[dstt_v.at[0]],
                    add=True)
    plsc.subcore_barrier()
    _dump_acc(acc, out_hbm, c, s)


@functools.partial(
    pl.kernel,
    out_type=jax.ShapeDtypeStruct((NC, N, D), jnp.float32),
    mesh=_MESH,
    scratch_types=[
        pltpu.VMEM((SRCR, 128), jnp.int32),
        pltpu.VMEM((SRCR, 128), jnp.int32),
        pltpu.VMEM((8, 16), jnp.int32),
        pltpu.VMEM((3 * CH, D), jnp.float32),
        pltpu.VMEM_SHARED((N, D), jnp.float32),
        pltpu.SemaphoreType.DMA,
        pltpu.SemaphoreType.DMA,
        pltpu.SemaphoreType.DMA,
        pltpu.SemaphoreType.DMA,
        pltpu.SemaphoreType.DMA,
        pltpu.SemaphoreType.DMA,
    ],
)
def _sc_scatter(xs_hbm, src_hbm, dst_hbm, dstt_hbm, zeros_hbm, out_hbm,
                src_v, dst_v, dstt_v, bufs, acc, g0, g1, g2, s0, s1, s2):
    c = lax.axis_index("c")
    s = lax.axis_index("s")
    wid = s * NC + c
    _init_acc(zeros_hbm, acc, s)
    pltpu.sync_copy(src_hbm.at[wid], src_v)
    pltpu.sync_copy(dst_hbm.at[wid], dst_v)
    pltpu.sync_copy(dstt_hbm.at[wid], dstt_v)
    plsc.subcore_barrier()

    gsem = (g0, g1, g2)
    ssem = (s0, s1, s2)

    def sidx(g):
        return src_v.at[g // 2, pl.ds(CH * (g % 2), CH)]

    def bufref(b):
        return bufs.at[pl.ds(CH * b, CH)]

    for b in range(3):
        pltpu.async_copy(xs_hbm.at[sidx(b)], bufref(b), gsem[b])

    def body(j, carry):
        for b in range(3):
            g = 3 * j + b
            pltpu.make_async_copy(xs_hbm.at[sidx(g)], bufref(b),
                                  gsem[b]).wait()
            pltpu.async_copy(bufref(b), acc.at[_didx(dst_v, g)], ssem[b],
                             add=True)
        for b in range(3):
            g = 3 * j + b
            pltpu.make_async_copy(bufref(b), acc.at[_didx(dst_v, g)],
                                  ssem[b]).wait()

            @pl.when(g + 3 < NFULL)
            def _():
                pltpu.async_copy(xs_hbm.at[sidx(g + 3)], bufref(b), gsem[b])

        return carry

    lax.fori_loop(0, NFULL // 3, body, 0)

    pltpu.sync_copy(xs_hbm.at[src_v.at[NFULL // 2, pl.ds(0, NTAIL)]],
                    bufs.at[pl.ds(0, NTAIL)])
    pltpu.sync_copy(bufs.at[pl.ds(0, NTAIL)], acc.at[dstt_v.at[0]], add=True)

    plsc.subcore_barrier()
    _dump_acc(acc, out_hbm, c, s)


def _prep_body(degp_ref, x_ref, dis_ref, xs_ref):
    deg3 = degp_ref[...]
    deg = (deg3[0] + deg3[1])[:, 0:1]
    dis = jnp.where(deg > 0, lax.rsqrt(jnp.maximum(deg, 1e-12)), 0.0)
    dis_ref[...] = dis
    xs_ref[...] = x_ref[...] * dis


_prep = pl.pallas_call(
    _prep_body,
    grid=(G,),
    in_specs=[
        pl.BlockSpec((NC, BR, D), lambda i: (0, i, 0)),
        pl.BlockSpec((BR, D), lambda i: (i, 0)),
    ],
    out_specs=[
        pl.BlockSpec((BR, 1), lambda i: (i, 0)),
        pl.BlockSpec((BR, D), lambda i: (i, 0)),
    ],
    out_shape=[
        jax.ShapeDtypeStruct((N, 1), jnp.float32),
        jax.ShapeDtypeStruct((N, D), jnp.float32),
    ],
)


def _lin_body(h_ref, w2_ref, b_ref, p_ref):
    p_ref[...] = (jnp.dot(h_ref[...], w2_ref[...],
                          preferred_element_type=jnp.float32) + b_ref[...])


_lin = pl.pallas_call(
    _lin_body,
    grid=(G,),
    in_specs=[
        pl.BlockSpec((BR, D), lambda i: (i, 0)),
        pl.BlockSpec((D, D), lambda i: (0, 0)),
        pl.BlockSpec((1, D), lambda i: (0, 0)),
    ],
    out_specs=pl.BlockSpec((BR, D), lambda i: (i, 0)),
    out_shape=jax.ShapeDtypeStruct((N, D), jnp.float32),
)


def _combine_body(aggp_ref, dis_ref, p_ref, w1_ref, out_ref, outs_ref):
    dis = dis_ref[...]
    agg3 = aggp_ref[...]
    agg = (agg3[0] + agg3[1]) * dis
    z = (jnp.dot(agg, w1_ref[...], preferred_element_type=jnp.float32)
         + p_ref[...])
    hh = jnp.where(z > 0, z, 0.2 * z)
    out_ref[...] = hh
    outs_ref[...] = hh * dis


_combine = pl.pallas_call(
    _combine_body,
    grid=(G,),
    in_specs=[
        pl.BlockSpec((NC, BR, D), lambda i: (0, i, 0)),
        pl.BlockSpec((BR, 1), lambda i: (i, 0)),
        pl.BlockSpec((BR, D), lambda i: (i, 0)),
        pl.BlockSpec((D, D), lambda i: (0, 0)),
    ],
    out_specs=[
        pl.BlockSpec((BR, D), lambda i: (i, 0)),
        pl.BlockSpec((BR, D), lambda i: (i, 0)),
    ],
    out_shape=[
        jax.ShapeDtypeStruct((N, D), jnp.float32),
        jax.ShapeDtypeStruct((N, D), jnp.float32),
    ],
)


def _final_body(aggp_ref, dis_ref, p_ref, w1_ref, wd_ref, bd_ref, out_ref):
    dis = dis_ref[...]
    agg3 = aggp_ref[...]
    agg = (agg3[0] + agg3[1]) * dis
    z = (jnp.dot(agg, w1_ref[...], preferred_element_type=jnp.float32)
         + p_ref[...])
    hh = jnp.where(z > 0, z, 0.2 * z)
    out_ref[...] = jnp.tanh(
        jnp.dot(hh, wd_ref[...], preferred_element_type=jnp.float32)
        + bd_ref[...])


_final = pl.pallas_call(
    _final_body,
    grid=(G,),
    in_specs=[
        pl.BlockSpec((NC, BR, D), lambda i: (0, i, 0)),
        pl.BlockSpec((BR, 1), lambda i: (i, 0)),
        pl.BlockSpec((BR, D), lambda i: (i, 0)),
        pl.BlockSpec((D, D), lambda i: (0, 0)),
        pl.BlockSpec((D, NLAB), lambda i: (0, 0)),
        pl.BlockSpec((1, NLAB), lambda i: (0, 0)),
    ],
    out_specs=pl.BlockSpec((BR, NLAB), lambda i: (i, 0)),
    out_shape=jax.ShapeDtypeStruct((N, NLAB), jnp.float32),
)


def kernel(x, edge_index, i, W1_1, W2_1, b_1, W1_2, W2_2, b_2,
           W1_3, W2_3, b_3, Wd, bd):
    src2 = edge_index[0].reshape(NW, EPW)
    dst2 = edge_index[1].reshape(NW, EPW)
    pad = EPAD - EPW
    src = jnp.pad(src2, ((0, 0), (0, pad))).reshape(NW, SRCR, 128)
    dst = jnp.pad(dst2, ((0, 0), (0, pad))).reshape(NW, SRCR, 128)
    dstt = jnp.pad(dst2[:, NFULL * CH:, None].reshape(NW, 1, NTAIL),
                   ((0, 0), (0, 7), (0, 0)))
    onesD = jnp.ones((CH, D), jnp.float32)
    zD = jnp.zeros((RPT, D), jnp.float32)

    degp = _sc_degree(dst, dstt, onesD, zD)
    p1 = _lin(x, W2_1, b_1.reshape(1, D))
    dis, xs = _prep(degp, x)

    aggp = _sc_scatter(xs, src, dst, dstt, zD)
    h1, h1s = _combine(aggp, dis, p1, W1_1)
    p2 = _lin(h1, W2_2, b_2.reshape(1, D))

    aggp = _sc_scatter(h1s, src, dst, dstt, zD)
    h2, h2s = _combine(aggp, dis, p2, W1_2)
    p3 = _lin(h2, W2_3, b_3.reshape(1, D))

    aggp = _sc_scatter(h2s, src, dst, dstt, zD)
    out = _final(aggp, dis, p3, W1_3, Wd, bd.reshape(1, NLAB))
    return out


# speedup vs baseline: 1.0134x; 1.0134x over previous
import functools

import jax
import jax.numpy as jnp
from jax import lax
from jax.experimental import pallas as pl
from jax.experimental.pallas import tpu as pltpu
from jax.experimental.pallas import tpu_sc as plsc

N = 10000
E = 320000
D = 128
NLAB = 10

NC = 2
NS = 16
NW = NC * NS
EPW = E // NW
EPAD = 10112
CH = 64
NFULL = EPW // CH
NTAIL = EPW - NFULL * CH
SRCR = EPAD // 128
RPT = 624
TAIL = N - NS * RPT

BR = 2000
G = N // BR

_MESH = plsc.VectorSubcoreMesh(core_axis_name="c", subcore_axis_name="s")


def _init_acc(zeros_hbm, acc, s):
    pltpu.sync_copy(zeros_hbm.at[pl.ds(0, RPT)], acc.at[pl.ds(s * RPT, RPT)])

    @pl.when(s == NS - 1)
    def _():
        pltpu.sync_copy(zeros_hbm.at[pl.ds(0, TAIL)],
                        acc.at[pl.ds(NS * RPT, TAIL)])


def _dump_acc(acc, out_hbm, c, s):
    pltpu.sync_copy(acc.at[pl.ds(s * RPT, RPT)],
                    out_hbm.at[c, pl.ds(s * RPT, RPT)])

    @pl.when(s == NS - 1)
    def _():
        pltpu.sync_copy(acc.at[pl.ds(NS * RPT, TAIL)],
                        out_hbm.at[c, pl.ds(NS * RPT, TAIL)])


def _didx(dst_v, g):
    return dst_v.at[g // 2, pl.ds(CH * (g % 2), CH)]


@functools.partial(
    pl.kernel,
    out_type=jax.ShapeDtypeStruct((NC, N, D), jnp.float32),
    mesh=_MESH,
    scratch_types=[
        pltpu.VMEM((SRCR, 128), jnp.int32),
        pltpu.VMEM((8, 16), jnp.int32),
        pltpu.VMEM((CH, D), jnp.float32),
        pltpu.VMEM_SHARED((N, D), jnp.float32),
        pltpu.SemaphoreType.DMA,
        pltpu.SemaphoreType.DMA,
        pltpu.SemaphoreType.DMA,
        pltpu.SemaphoreType.DMA,
    ],
)
def _sc_degree(dst_hbm, dstt_hbm, ones_hbm, zeros_hbm, out_hbm,
               dst_v, dstt_v, ones_v, acc, s0, s1, s2, s3):
    c = lax.axis_index("c")
    s = lax.axis_index("s")
    wid = s * NC + c
    _init_acc(zeros_hbm, acc, s)
    pltpu.sync_copy(dst_hbm.at[wid], dst_v)
    pltpu.sync_copy(dstt_hbm.at[wid], dstt_v)
    pltpu.sync_copy(ones_hbm, ones_v)
    plsc.subcore_barrier()

    sems = (s0, s1, s2, s3)

    def body(j, carry):
        for b in range(4):
            pltpu.async_copy(ones_v, acc.at[_didx(dst_v, 4 * j + b)],
                             sems[b], add=True)
        for b in range(4):
            pltpu.make_async_copy(ones_v, acc.at[_didx(dst_v, 4 * j + b)],
                                  sems[b]).wait()
        return carry

    lax.fori_loop(0, NFULL // 4, body, 0)
    pltpu.sync_copy(ones_v.at[pl.ds(0, NTAIL)], acc.at[dstt_v.at[0]],
                    add=True)
    plsc.subcore_barrier()
    _dump_acc(acc, out_hbm, c, s)


@functools.partial(
    pl.kernel,
    out_type=jax.ShapeDtypeStruct((NC, N, D), jnp.float32),
    mesh=_MESH,
    scratch_types=[
        pltpu.VMEM((SRCR, 128), jnp.int32),
        pltpu.VMEM((SRCR, 128), jnp.int32),
        pltpu.VMEM((8, 16), jnp.int32),
        pltpu.VMEM((3 * CH, D), jnp.float32),
        pltpu.VMEM_SHARED((N, D), jnp.float32),
        pltpu.SemaphoreType.DMA,
        pltpu.SemaphoreType.DMA,
        pltpu.SemaphoreType.DMA,
        pltpu.SemaphoreType.DMA,
        pltpu.SemaphoreType.DMA,
        pltpu.SemaphoreType.DMA,
    ],
)
def _sc_scatter(xs_hbm, src_hbm, dst_hbm, dstt_hbm, zeros_hbm, out_hbm,
                src_v, dst_v, dstt_v, bufs, acc, g0, g1, g2, s0, s1, s2):
    c = lax.axis_index("c")
    s = lax.axis_index("s")
    wid = s * NC + c
    _init_acc(zeros_hbm, acc, s)
    pltpu.sync_copy(src_hbm.at[wid], src_v)
    pltpu.sync_copy(dst_hbm.at[wid], dst_v)
    pltpu.sync_copy(dstt_hbm.at[wid], dstt_v)
    plsc.subcore_barrier()

    gsem = (g0, g1, g2)
    ssem = (s0, s1, s2)

    def sidx(g):
        return src_v.at[g // 2, pl.ds(CH * (g % 2), CH)]

    def bufref(b):
        return bufs.at[pl.ds(CH * b, CH)]

    for b in range(3):
        pltpu.async_copy(xs_hbm.at[sidx(b)], bufref(b), gsem[b])

    def body(j, carry):
        for b in range(3):
            g = 3 * j + b
            pltpu.make_async_copy(xs_hbm.at[sidx(g)], bufref(b),
                                  gsem[b]).wait()
            pltpu.async_copy(bufref(b), acc.at[_didx(dst_v, g)], ssem[b],
                             add=True)
        for b in range(3):
            g = 3 * j + b
            pltpu.make_async_copy(bufref(b), acc.at[_didx(dst_v, g)],
                                  ssem[b]).wait()

            @pl.when(g + 3 < NFULL)
            def _():
                pltpu.async_copy(xs_hbm.at[sidx(g + 3)], bufref(b), gsem[b])

        return carry

    lax.fori_loop(0, NFULL // 3, body, 0)

    pltpu.sync_copy(xs_hbm.at[src_v.at[NFULL // 2, pl.ds(0, NTAIL)]],
                    bufs.at[pl.ds(0, NTAIL)])
    pltpu.sync_copy(bufs.at[pl.ds(0, NTAIL)], acc.at[dstt_v.at[0]], add=True)

    plsc.subcore_barrier()
    _dump_acc(acc, out_hbm, c, s)


def _prep_body(degp_ref, x_ref, dis_ref, xs_ref):
    deg3 = degp_ref[...]
    deg = (deg3[0] + deg3[1])[:, 0:1]
    dis = jnp.where(deg > 0, lax.rsqrt(jnp.maximum(deg, 1e-12)), 0.0)
    dis_ref[...] = dis
    xs_ref[...] = x_ref[...] * dis


_prep = pl.pallas_call(
    _prep_body,
    grid=(G,),
    in_specs=[
        pl.BlockSpec((NC, BR, D), lambda i: (0, i, 0)),
        pl.BlockSpec((BR, D), lambda i: (i, 0)),
    ],
    out_specs=[
        pl.BlockSpec((BR, 1), lambda i: (i, 0)),
        pl.BlockSpec((BR, D), lambda i: (i, 0)),
    ],
    out_shape=[
        jax.ShapeDtypeStruct((N, 1), jnp.float32),
        jax.ShapeDtypeStruct((N, D), jnp.float32),
    ],
)


def _lin_body(h_ref, w2_ref, b_ref, p_ref):
    p_ref[...] = (jnp.dot(h_ref[...], w2_ref[...],
                          preferred_element_type=jnp.float32) + b_ref[...])


_lin = pl.pallas_call(
    _lin_body,
    grid=(G,),
    in_specs=[
        pl.BlockSpec((BR, D), lambda i: (i, 0)),
        pl.BlockSpec((D, D), lambda i: (0, 0)),
        pl.BlockSpec((1, D), lambda i: (0, 0)),
    ],
    out_specs=pl.BlockSpec((BR, D), lambda i: (i, 0)),
    out_shape=jax.ShapeDtypeStruct((N, D), jnp.float32),
)


def _combine_body(aggp_ref, dis_ref, p_ref, w1_ref, out_ref, outs_ref):
    dis = dis_ref[...]
    agg3 = aggp_ref[...]
    agg = (agg3[0] + agg3[1]) * dis
    z = (jnp.dot(agg, w1_ref[...], preferred_element_type=jnp.float32)
         + p_ref[...])
    hh = jnp.where(z > 0, z, 0.2 * z)
    out_ref[...] = hh
    outs_ref[...] = hh * dis


_combine = pl.pallas_call(
    _combine_body,
    grid=(G,),
    in_specs=[
        pl.BlockSpec((NC, BR, D), lambda i: (0, i, 0)),
        pl.BlockSpec((BR, 1), lambda i: (i, 0)),
        pl.BlockSpec((BR, D), lambda i: (i, 0)),
        pl.BlockSpec((D, D), lambda i: (0, 0)),
    ],
    out_specs=[
        pl.BlockSpec((BR, D), lambda i: (i, 0)),
        pl.BlockSpec((BR, D), lambda i: (i, 0)),
    ],
    out_shape=[
        jax.ShapeDtypeStruct((N, D), jnp.float32),
        jax.ShapeDtypeStruct((N, D), jnp.float32),
    ],
)


def _final_body(aggp_ref, dis_ref, p_ref, w1_ref, wd_ref, bd_ref, out_ref):
    dis = dis_ref[...]
    agg3 = aggp_ref[...]
    agg = (agg3[0] + agg3[1]) * dis
    z = (jnp.dot(agg, w1_ref[...], preferred_element_type=jnp.float32)
         + p_ref[...])
    hh = jnp.where(z > 0, z, 0.2 * z)
    out_ref[...] = jnp.tanh(
        jnp.dot(hh, wd_ref[...], preferred_element_type=jnp.float32)
        + bd_ref[...])


_final = pl.pallas_call(
    _final_body,
    grid=(G,),
    in_specs=[
        pl.BlockSpec((NC, BR, D), lambda i: (0, i, 0)),
        pl.BlockSpec((BR, 1), lambda i: (i, 0)),
        pl.BlockSpec((BR, D), lambda i: (i, 0)),
        pl.BlockSpec((D, D), lambda i: (0, 0)),
        pl.BlockSpec((D, NLAB), lambda i: (0, 0)),
        pl.BlockSpec((1, NLAB), lambda i: (0, 0)),
    ],
    out_specs=pl.BlockSpec((BR, NLAB), lambda i: (i, 0)),
    out_shape=jax.ShapeDtypeStruct((N, NLAB), jnp.float32),
)


def kernel(x, edge_index, i, W1_1, W2_1, b_1, W1_2, W2_2, b_2,
           W1_3, W2_3, b_3, Wd, bd):
    src2 = edge_index[0].reshape(NW, EPW)
    dst2 = edge_index[1].reshape(NW, EPW)
    pad = EPAD - EPW
    src = jnp.pad(src2, ((0, 0), (0, pad))).reshape(NW, SRCR, 128)
    dst = jnp.pad(dst2, ((0, 0), (0, pad))).reshape(NW, SRCR, 128)
    dstt = jnp.pad(dst2[:, NFULL * CH:, None].reshape(NW, 1, NTAIL),
                   ((0, 0), (0, 7), (0, 0)))
    onesD = jnp.ones((CH, D), jnp.float32)
    zD = jnp.zeros((RPT, D), jnp.float32)

    degp = _sc_degree(dst, dstt, onesD, zD)
    p1 = _lin(x, W2_1, b_1.reshape(1, D))
    dis, xs = _prep(degp, x)

    aggp = _sc_scatter(xs, src, dst, dstt, zD)
    h1, h1s = _combine(aggp, dis, p1, W1_1)
    p2 = _lin(h1, W2_2, b_2.reshape(1, D))

    aggp = _sc_scatter(h1s, src, dst, dstt, zD)
    h2, h2s = _combine(aggp, dis, p2, W1_2)
    p3 = _lin(h2, W2_3, b_3.reshape(1, D))

    aggp = _sc_scatter(h2s, src, dst, dstt, zD)
    out = _final(aggp, dis, p3, W1_3, Wd, bd.reshape(1, NLAB))
    return out
